# baseline (speedup 1.0000x reference)
import dataclasses
import functools

import jax
import jax.numpy as jnp
from jax import lax
from jax.experimental import pallas as pl
from jax.experimental.pallas import tpu as pltpu
from jax.experimental.pallas import tpu_sc as plsc

B = 16384
F = 64
NC = 2
NS = 16
NW = NC * NS
BPW = B // NW
L = 16


def _sc_pairmf(user, item_i, item_j, embed_user, embed_item):
    mesh = plsc.VectorSubcoreMesh(core_axis_name="c", subcore_axis_name="s")
    cp = pltpu.CompilerParams(
        needs_layout_passes=False, use_tc_tiling_on_sc=False
    )
    out_type = (
        jax.ShapeDtypeStruct((B,), jnp.float32),
        jax.ShapeDtypeStruct((B,), jnp.float32),
    )

    @functools.partial(
        pl.kernel,
        out_type=out_type,
        mesh=mesh,
        compiler_params=cp,
        scratch_types=[
            pltpu.VMEM((BPW,), jnp.int32),
            pltpu.VMEM((BPW,), jnp.int32),
            pltpu.VMEM((BPW,), jnp.int32),
            pltpu.VMEM((BPW, F), jnp.bfloat16),
            pltpu.VMEM((BPW, F), jnp.bfloat16),
            pltpu.VMEM((BPW, F), jnp.bfloat16),
            pltpu.VMEM((BPW,), jnp.float32),
            pltpu.VMEM((BPW,), jnp.float32),
            pltpu.SemaphoreType.DMA,
            pltpu.SemaphoreType.DMA,
            pltpu.SemaphoreType.DMA,
        ],
    )
    def k(user_hbm, ii_hbm, ij_hbm, eu_hbm, ei_hbm, oi_hbm, oj_hbm,
          uidx, iidx, jidx, urows, irows, jrows, oi_v, oj_v, su, si, sj):
        wid = lax.axis_index("s") * NC + lax.axis_index("c")
        base = wid * BPW

        pltpu.sync_copy(user_hbm.at[pl.ds(base, BPW)], uidx)
        pltpu.sync_copy(ii_hbm.at[pl.ds(base, BPW)], iidx)
        pltpu.sync_copy(ij_hbm.at[pl.ds(base, BPW)], jidx)

        cu = pltpu.async_copy(eu_hbm.at[uidx], urows, su)
        ci = pltpu.async_copy(ei_hbm.at[iidx], irows, si)
        cj = pltpu.async_copy(ei_hbm.at[jidx], jrows, sj)
        cu.wait()
        ci.wait()
        cj.wait()

        lane = lax.iota(jnp.int32, L)
        m15 = lane == (L - 1)

        @pl.loop(0, BPW)
        def _(r):
            u0, u1 = plsc.unpack(urows[r, pl.ds(0, 2 * L)],
                                 format=plsc.PackFormat.INTERLEAVED)
            u2, u3 = plsc.unpack(urows[r, pl.ds(2 * L, 2 * L)],
                                 format=plsc.PackFormat.INTERLEAVED)
            a0, a1 = plsc.unpack(irows[r, pl.ds(0, 2 * L)],
                                 format=plsc.PackFormat.INTERLEAVED)
            a2, a3 = plsc.unpack(irows[r, pl.ds(2 * L, 2 * L)],
                                 format=plsc.PackFormat.INTERLEAVED)
            b0, b1 = plsc.unpack(jrows[r, pl.ds(0, 2 * L)],
                                 format=plsc.PackFormat.INTERLEAVED)
            b2, b3 = plsc.unpack(jrows[r, pl.ds(2 * L, 2 * L)],
                                 format=plsc.PackFormat.INTERLEAVED)
            acc_i = u0 * a0 + u1 * a1 + u2 * a2 + u3 * a3
            acc_j = u0 * b0 + u1 * b1 + u2 * b2 + u3 * b3
            ridx = jnp.full((L,), r, jnp.int32)
            plsc.store_scatter(oi_v, [ridx], plsc.cumsum(acc_i), mask=m15)
            plsc.store_scatter(oj_v, [ridx], plsc.cumsum(acc_j), mask=m15)

        pltpu.sync_copy(oi_v, oi_hbm.at[pl.ds(base, BPW)])
        pltpu.sync_copy(oj_v, oj_hbm.at[pl.ds(base, BPW)])

    return k(user, item_i, item_j, embed_user, embed_item)


def kernel(user, item_i, item_j, embed_user, embed_item):
    user = user.astype(jnp.int32)
    item_i = item_i.astype(jnp.int32)
    item_j = item_j.astype(jnp.int32)
    embed_user = embed_user.astype(jnp.bfloat16)
    embed_item = embed_item.astype(jnp.bfloat16)
    return _sc_pairmf(user, item_i, item_j, embed_user, embed_item)


# speedup vs baseline: 2.1655x; 2.1655x over previous
import functools

import jax
import jax.numpy as jnp
from jax import lax
from jax.experimental import pallas as pl
from jax.experimental.pallas import tpu as pltpu
from jax.experimental.pallas import tpu_sc as plsc

B = 16384
F = 64
NC = 2
NS = 16
NW = NC * NS
BPW = B // NW
L = 16
BLK = 128


def _sc_pairmf(user, item_i, item_j, eu_t, ei_t):
    mesh = plsc.VectorSubcoreMesh(core_axis_name="c", subcore_axis_name="s")
    cp = pltpu.CompilerParams(
        needs_layout_passes=False,
        use_tc_tiling_on_sc=True,
        disable_bounds_checks=True,
    )
    out_type = (
        jax.ShapeDtypeStruct((B,), jnp.float32),
        jax.ShapeDtypeStruct((B,), jnp.float32),
    )
    blk = pltpu.VMEM((F, BLK), jnp.float32)

    @functools.partial(
        pl.kernel,
        out_type=out_type,
        mesh=mesh,
        compiler_params=cp,
        scratch_types=[
            pltpu.VMEM((BPW,), jnp.int32),
            pltpu.VMEM((BPW,), jnp.int32),
            pltpu.VMEM((BPW,), jnp.int32),
            [[blk, blk], [blk, blk], [blk, blk]],
            [[blk, blk], [blk, blk], [blk, blk]],
            pltpu.VMEM((BPW,), jnp.float32),
            pltpu.VMEM((BPW,), jnp.float32),
            pltpu.SemaphoreType.DMA,
            pltpu.SemaphoreType.DMA,
        ],
    )
    def k(user_hbm, ii_hbm, ij_hbm, eu_hbm, ei_hbm, oi_hbm, oj_hbm,
          idx_u, idx_i, idx_j, bufs0, bufs1, oi_v, oj_v, sem0, sem1):
        wid = lax.axis_index("s") * NC + lax.axis_index("c")
        base = wid * BPW

        pltpu.sync_copy(user_hbm.at[pl.ds(base, BPW)], idx_u)
        pltpu.sync_copy(ii_hbm.at[pl.ds(base, BPW)], idx_i)
        pltpu.sync_copy(ij_hbm.at[pl.ds(base, BPW)], idx_j)

        lane = lax.iota(jnp.int32, L)
        m15 = lane == (L - 1)
        zero16 = jnp.zeros((L,), jnp.int32)
        bufs = (bufs0, bufs1)
        sems = (sem0, sem1)
        tables = (eu_hbm, ei_hbm, ei_hbm)
        idxs = (idx_u, idx_i, idx_j)

        def extract(idx_v, r):
            chunk = idx_v[pl.ds((r // L) * L, L)]
            return jnp.sum(jnp.where(lane == (r % L), chunk, zero16))

        def issue(row, phase, slot, sem):
            for t in range(3):
                v = extract(idxs[t], row)
                off = pl.multiple_of(
                    lax.shift_right_logical(v, 7) * BLK, BLK)
                pltpu.async_copy(
                    tables[t].at[:, pl.ds(off, BLK)],
                    bufs[phase][t][slot], sem)

        def drain(phase, slot, sem):
            for t in range(3):
                pltpu.make_async_copy(
                    tables[t].at[:, pl.ds(0, BLK)],
                    bufs[phase][t][slot], sem).wait()

        def compute(row, phase, slot):
            cs = [lax.bitwise_and(extract(idxs[t], row), BLK - 1)
                  for t in range(3)]
            cvecs = [jnp.full((L,), c, jnp.int32) for c in cs]
            ub, ib, jb = (bufs[phase][t][slot] for t in range(3))
            acc_i = jnp.zeros((L,), jnp.float32)
            acc_j = jnp.zeros((L,), jnp.float32)
            for g in range(4):
                fvec = lane + g * L
                u = plsc.load_gather(ub, [fvec, cvecs[0]])
                acc_i = acc_i + u * plsc.load_gather(ib, [fvec, cvecs[1]])
                acc_j = acc_j + u * plsc.load_gather(jb, [fvec, cvecs[2]])
            rvec = jnp.full((L,), row, jnp.int32)
            plsc.store_scatter(oi_v, [rvec], plsc.cumsum(acc_i), mask=m15)
            plsc.store_scatter(oj_v, [rvec], plsc.cumsum(acc_j), mask=m15)

        for p in range(2):
            for s in range(2):
                issue(2 * p + s, p, s, sems[p])

        @pl.loop(0, BPW // 4)
        def _(kk):
            r0 = kk * 4
            for p in range(2):
                for s in range(2):
                    drain(p, s, sems[p])
                for s in range(2):
                    compute(r0 + 2 * p + s, p, s)

                @pl.when(kk < BPW // 4 - 1)
                def _():
                    for s in range(2):
                        issue(r0 + 4 + 2 * p + s, p, s, sems[p])

        pltpu.sync_copy(oi_v, oi_hbm.at[pl.ds(base, BPW)])
        pltpu.sync_copy(oj_v, oj_hbm.at[pl.ds(base, BPW)])

    return k(user, item_i, item_j, eu_t, ei_t)


def kernel(user, item_i, item_j, embed_user, embed_item):
    user = user.astype(jnp.int32)
    item_i = item_i.astype(jnp.int32)
    item_j = item_j.astype(jnp.int32)
    return _sc_pairmf(user, item_i, item_j, embed_user.T, embed_item.T)
